# baseline (speedup 1.0000x reference)
import jax
import jax.numpy as jnp
from jax import lax
from jax.experimental import pallas as pl
from jax.experimental.pallas import tpu as pltpu
from jax.experimental.pallas import tpu_sc as plsc

ROWS = 16384
COLS = 2048
LANES_SC = 16
SUB = COLS // LANES_SC
NUM_WORKERS = 32
ROWS_PER_W = ROWS // NUM_WORKERS
CHUNK_ROWS = 8
CHUNK_ELEMS = CHUNK_ROWS * COLS
N_CHUNKS = ROWS_PER_W // CHUNK_ROWS


def _sc_flip(in_hbm, out_hbm, in_v, out_v):
    c = lax.axis_index("c")
    s = lax.axis_index("s")
    wid = s * 2 + c
    base = wid * (ROWS_PER_W * COLS)

    def chunk_body(ci, carry):
        off = base + ci * CHUNK_ELEMS
        pltpu.sync_copy(in_hbm.at[pl.ds(off, CHUNK_ELEMS)], in_v)

        def row_body(r, carry2):
            rbase = r * COLS
            for k in range(SUB):
                x = in_v[pl.ds(rbase + (SUB - 1 - k) * LANES_SC, LANES_SC)]
                out_v[pl.ds(rbase + k * LANES_SC, LANES_SC)] = jnp.flip(
                    x, axis=0
                )
            return carry2

        lax.fori_loop(0, CHUNK_ROWS, row_body, 0)
        pltpu.sync_copy(out_v, out_hbm.at[pl.ds(off, CHUNK_ELEMS)])
        return carry

    lax.fori_loop(0, N_CHUNKS, chunk_body, 0)


def kernel(inputs, cond_inputs):
    flat_in = inputs.reshape(ROWS * COLS)
    mesh = plsc.VectorSubcoreMesh(core_axis_name="c", subcore_axis_name="s")
    f = pl.kernel(
        _sc_flip,
        mesh=mesh,
        out_type=jax.ShapeDtypeStruct((ROWS * COLS,), jnp.float32),
        scratch_types=[
            pltpu.VMEM((CHUNK_ELEMS,), jnp.float32),
            pltpu.VMEM((CHUNK_ELEMS,), jnp.float32),
        ],
    )
    out = f(flat_in)
    return (out.reshape(ROWS, COLS), 0.0)
